# Initial kernel scaffold; baseline (speedup 1.0000x reference)
import jax
import jax.numpy as jnp
from jax.experimental import pallas as pl


def kernel(pos):
    raise NotImplementedError("write your pallas kernel here")


# speedup vs baseline: 5.2414x; 5.2414x over previous
import functools

import jax
import jax.numpy as jnp
from jax.experimental import pallas as pl
from jax.experimental.pallas import tpu as pltpu

_N = 10000
_NODE_MASKING = 0.3
_RADIUS = 0.75
_K = 50
_NUM_BINS = 5
_CUTOFF = 5.0
_N_KEEP = int(_N * (1.0 - _NODE_MASKING))

_BR = 256
_PAD = 7168
_KPAD = 64
_F = 128


def _augment_consts():
    base = jax.random.key(1)
    k1 = jax.random.fold_in(base, 0)
    k2 = jax.random.fold_in(base, 1)
    k3 = jax.random.fold_in(base, 2)
    scores = jax.random.uniform(k1, (_N,))
    keep_idx = jnp.argsort(scores)[:_N_KEEP]
    dirs = jax.random.normal(k2, (_N_KEEP, 3), dtype=jnp.float32)
    dirs = dirs / (jnp.linalg.norm(dirs, axis=1, keepdims=True) + 1e-12)
    u = jax.random.uniform(k3, (_N_KEEP, 1), dtype=jnp.float32)
    noise = dirs * _RADIUS * (u ** (1.0 / 3.0))
    return keep_idx, noise


def _knn_rbf_kernel(a_ref, bt_ref, btb_ref, sqc_ref, nbr_ref, ea_ref,
                    sel_ref, ex_ref):
    i = pl.program_id(0)
    a = a_ref[...]
    sq_r = jnp.sum(a * a, axis=1, keepdims=True)
    dot = jnp.dot(a.astype(jnp.bfloat16), btb_ref[...],
                  preferred_element_type=jnp.float32)
    d2s = sq_r + sqc_ref[0:1, :] - 2.0 * dot
    d2e = jnp.zeros((_BR, _PAD), jnp.float32)
    for f in range(3):
        d2e = d2e + (a[:, f:f + 1] - bt_ref[f:f + 1, :]) ** 2
    col = jax.lax.broadcasted_iota(jnp.int32, d2s.shape, 1)
    row = jax.lax.broadcasted_iota(jnp.int32, d2s.shape, 0) + i * _BR
    d2s = jnp.where((col == row) | (col >= _N_KEEP), jnp.inf, d2s)
    sel_ref[...] = d2s
    ex_ref[...] = d2e

    kcol = jax.lax.broadcasted_iota(jnp.int32, (_BR, _KPAD), 1)

    def body(t, carry):
        nbr_acc, val_acc = carry
        v = sel_ref[...]
        m = jnp.min(v, axis=1, keepdims=True)
        idx = jnp.min(jnp.where(v == m, col, _PAD), axis=1,
                      keepdims=True)
        hit = col == idx
        ev = jnp.min(jnp.where(hit, ex_ref[...], jnp.inf), axis=1,
                     keepdims=True)
        sel_ref[...] = jnp.where(hit, jnp.inf, v)
        sel = kcol == t
        nbr_acc = jnp.where(sel, idx, nbr_acc)
        val_acc = jnp.where(sel, ev, val_acc)
        return nbr_acc, val_acc

    nbr0 = jnp.zeros((_BR, _KPAD), jnp.int32)
    val0 = jnp.zeros((_BR, _KPAD), jnp.float32)
    nbr_acc, val_acc = jax.lax.fori_loop(0, _K, body, (nbr0, val0))
    nbr_ref[...] = nbr_acc

    dist = jnp.sqrt(val_acc + 1e-12)
    sigma = _CUTOFF / (_NUM_BINS - 1)
    for b in range(_NUM_BINS):
        c = b * sigma
        ea_ref[:, b * _KPAD:(b + 1) * _KPAD] = jnp.exp(
            -((dist - c) ** 2) / (2.0 * sigma * sigma))


@functools.partial(jax.jit, static_argnums=())
def kernel(pos):
    keep_idx, noise = _augment_consts()
    p = jnp.take(pos, keep_idx, axis=0) + noise

    p_pad = jnp.zeros((_PAD, _F), jnp.float32).at[:_N_KEEP, :3].set(p)
    bt = p_pad.T
    btb = bt.astype(jnp.bfloat16)
    sqc = jnp.broadcast_to(jnp.sum(p_pad * p_pad, axis=1)[None, :], (8, _PAD))

    grid = _PAD // _BR
    nbr_full, ea_full = pl.pallas_call(
        _knn_rbf_kernel,
        grid=(grid,),
        in_specs=[
            pl.BlockSpec((_BR, _F), lambda i: (i, 0)),
            pl.BlockSpec((_F, _PAD), lambda i: (0, 0)),
            pl.BlockSpec((_F, _PAD), lambda i: (0, 0)),
            pl.BlockSpec((8, _PAD), lambda i: (0, 0)),
        ],
        out_specs=[
            pl.BlockSpec((_BR, _KPAD), lambda i: (i, 0)),
            pl.BlockSpec((_BR, _NUM_BINS * _KPAD), lambda i: (i, 0)),
        ],
        out_shape=[
            jax.ShapeDtypeStruct((_PAD, _KPAD), jnp.int32),
            jax.ShapeDtypeStruct((_PAD, _NUM_BINS * _KPAD), jnp.float32),
        ],
        scratch_shapes=[pltpu.VMEM((_BR, _PAD), jnp.float32),
                        pltpu.VMEM((_BR, _PAD), jnp.float32)],
    )(p_pad, bt, btb, sqc)

    nbr = nbr_full[:_N_KEEP, :_K]
    ea = (ea_full[:_N_KEEP]
          .reshape(_N_KEEP, _NUM_BINS, _KPAD)[:, :, :_K]
          .transpose(0, 2, 1)
          .reshape(_N_KEEP * _K, _NUM_BINS))
    edge_attr = jnp.concatenate([ea, ea], axis=0)

    dst = jnp.repeat(jnp.arange(_N_KEEP, dtype=nbr.dtype), _K)
    src = nbr.reshape(-1)
    edge_index = jnp.stack(
        [jnp.concatenate([src, dst]), jnp.concatenate([dst, src])], axis=0)
    return edge_index, edge_attr
